# Initial kernel scaffold; baseline (speedup 1.0000x reference)
import jax
import jax.numpy as jnp
from jax.experimental import pallas as pl


def kernel(urls, ranks, vtypes, q_iter, url_table, rank_table, vtype_table, qcnt_table, W, b):
    raise NotImplementedError("write your pallas kernel here")


# speedup vs baseline: 8.8925x; 8.8925x over previous
import functools

import jax
import jax.numpy as jnp
from jax import lax
from jax.experimental import pallas as pl
from jax.experimental.pallas import tpu as pltpu
from jax.experimental.pallas import tpu_sc as plsc

NC, NS = 2, 16
NW = NC * NS
IDX_W = 128
T = 512
MT = 1024


def _make_sc_gather(n_tokens, d):
    tpw = n_tokens // NW
    steps = tpw // T
    n_rows = n_tokens // IDX_W
    mesh = plsc.VectorSubcoreMesh(core_axis_name="c", subcore_axis_name="s")

    @functools.partial(
        pl.kernel,
        out_type=jax.ShapeDtypeStruct((n_rows, IDX_W, d), jnp.float32),
        mesh=mesh,
        compiler_params=pltpu.CompilerParams(use_tc_tiling_on_sc=False),
        scratch_types=[
            pltpu.VMEM((T // IDX_W, IDX_W), jnp.int32),
            pltpu.VMEM((T // IDX_W, IDX_W, d), jnp.float32),
            pltpu.SemaphoreType.DMA,
        ],
    )
    def sc_gather(idx_hbm, table_hbm, out_hbm, idx_v, rows_v, sem):
        cid = lax.axis_index("c")
        sid = lax.axis_index("s")
        wid = sid * NC + cid
        row_base = wid * (tpw // IDX_W)

        def body(step, carry):
            row_off = row_base + step * (T // IDX_W)
            pltpu.sync_copy(idx_hbm.at[pl.ds(row_off, T // IDX_W)], idx_v)
            cps = [
                pltpu.async_copy(table_hbm.at[idx_v.at[j]], rows_v.at[j], sem)
                for j in range(T // IDX_W)
            ]
            for cp in cps:
                cp.wait()
            pltpu.sync_copy(rows_v, out_hbm.at[pl.ds(row_off, T // IDX_W)])
            return carry

        lax.fori_loop(0, steps, body, 0)

    return sc_gather


def _tc_body(g_ref, r_ref, v_ref, q_ref, w_ref, b_ref, cat_ref, o_ref):
    w = w_ref[...]
    w_url = w[0:64, :]
    w_small = w[64:80, :]
    small_proj = jnp.dot(cat_ref[...], w_small,
                         preferred_element_type=jnp.float32)
    r = r_ref[0]
    v = v_ref[0]
    q = q_ref[0]
    ji = lax.broadcasted_iota(jnp.int32, (128, MT), 0)
    mh = (ji == r) | (ji == v + 11) | (ji == q + 111)
    mhf = jnp.where(mh, 1.0, 0.0).astype(jnp.float32)
    small = lax.dot_general(mhf, small_proj, (((0,), (0,)), ((), ())),
                            preferred_element_type=jnp.float32)
    url_part = jnp.dot(g_ref[...], w_url,
                       preferred_element_type=jnp.float32)
    o_ref[...] = jnp.tanh(url_part + small + b_ref[...])


def _tc_call(g, r3, v3, q3, w, b2, small_cat):
    n = g.shape[0]
    nb = n // MT
    return pl.pallas_call(
        _tc_body,
        grid=(nb,),
        in_specs=[
            pl.BlockSpec((MT, g.shape[1]), lambda i: (i, 0)),
            pl.BlockSpec((1, 1, MT), lambda i: (i, 0, 0)),
            pl.BlockSpec((1, 1, MT), lambda i: (i, 0, 0)),
            pl.BlockSpec((1, 1, MT), lambda i: (i, 0, 0)),
            pl.BlockSpec((80, 128), lambda i: (0, 0)),
            pl.BlockSpec((1, 128), lambda i: (0, 0)),
            pl.BlockSpec((128, 16), lambda i: (0, 0)),
        ],
        out_specs=pl.BlockSpec((MT, 128), lambda i: (i, 0)),
        out_shape=jax.ShapeDtypeStruct((n, 128), jnp.float32),
    )(g, r3, v3, q3, w, b2, small_cat)


def kernel(urls, ranks, vtypes, q_iter, url_table, rank_table, vtype_table,
           qcnt_table, W, b):
    B, L = urls.shape
    n = B * L
    d = url_table.shape[1]
    idx2 = urls.reshape(n // IDX_W, IDX_W).astype(jnp.int32)
    g = _make_sc_gather(n, d)(idx2, url_table).reshape(n, d)

    small_cat = jnp.zeros((128, 16), jnp.float32)
    small_cat = small_cat.at[0:11, 0:4].set(rank_table)
    small_cat = small_cat.at[11:111, 4:12].set(vtype_table)
    small_cat = small_cat.at[111:122, 12:16].set(qcnt_table)

    nb = n // MT
    r3 = ranks.reshape(nb, 1, MT).astype(jnp.int32)
    v3 = vtypes.reshape(nb, 1, MT).astype(jnp.int32)
    q3 = q_iter.reshape(nb, 1, MT).astype(jnp.int32)
    out = _tc_call(g, r3, v3, q3, W, b.reshape(1, 128), small_cat)
    return out.reshape(B, L, 128)


# speedup vs baseline: 10.1606x; 1.1426x over previous
import functools

import jax
import jax.numpy as jnp
from jax import lax
from jax.experimental import pallas as pl
from jax.experimental.pallas import tpu as pltpu
from jax.experimental.pallas import tpu_sc as plsc

NC, NS = 2, 16
NW = NC * NS
IDX_W = 128
T = 512
MT = 1600


def _make_sc_gather(n_tokens, d):
    tpw = n_tokens // NW
    steps = tpw // T
    n_rows = n_tokens // IDX_W
    mesh = plsc.VectorSubcoreMesh(core_axis_name="c", subcore_axis_name="s")

    @functools.partial(
        pl.kernel,
        out_type=jax.ShapeDtypeStruct((n_tokens, d), jnp.float32),
        mesh=mesh,
        compiler_params=pltpu.CompilerParams(use_tc_tiling_on_sc=False),
        scratch_types=[
            pltpu.VMEM((T // IDX_W, IDX_W), jnp.int32),
            pltpu.VMEM((T, d), jnp.float32),
            pltpu.SemaphoreType.DMA,
        ],
    )
    def sc_gather(idx_hbm, table_hbm, out_hbm, idx_v, rows_v, sem):
        cid = lax.axis_index("c")
        sid = lax.axis_index("s")
        wid = sid * NC + cid
        row_base = wid * (tpw // IDX_W)

        def body(step, carry):
            row_off = row_base + step * (T // IDX_W)
            pltpu.sync_copy(idx_hbm.at[pl.ds(row_off, T // IDX_W)], idx_v)
            cps = [
                pltpu.async_copy(table_hbm.at[idx_v.at[j]],
                                 rows_v.at[pl.ds(j * IDX_W, IDX_W)], sem)
                for j in range(T // IDX_W)
            ]
            for cp in cps:
                cp.wait()
            pltpu.sync_copy(rows_v, out_hbm.at[pl.ds(row_off * IDX_W, T)])
            return carry

        lax.fori_loop(0, steps, body, 0)

    return sc_gather


def _tc_body(g_ref, r_ref, v_ref, q_ref, w_ref, b_ref, cat_ref, o_ref):
    w = w_ref[...]
    w_url = w[0:64, :]
    w_small = w[64:80, :]
    small_proj = jnp.dot(cat_ref[...], w_small,
                         preferred_element_type=jnp.float32)
    r = r_ref[0]
    v = v_ref[0]
    q = q_ref[0]
    ji = lax.broadcasted_iota(jnp.int32, (128, MT), 0)
    mh = (ji == r) | (ji == v + 11) | (ji == q + 111)
    mhf = jnp.where(mh, 1.0, 0.0).astype(jnp.float32)
    small = lax.dot_general(mhf, small_proj, (((0,), (0,)), ((), ())),
                            preferred_element_type=jnp.float32)
    url_part = jnp.dot(g_ref[...], w_url,
                       preferred_element_type=jnp.float32)
    res = jnp.tanh(url_part + small + b_ref[...])
    o_ref[...] = res.reshape(o_ref.shape)


def _tc_call(g, r3, v3, q3, w, b2, small_cat, B, L):
    n = g.shape[0]
    nb = n // MT
    mb = MT // L
    return pl.pallas_call(
        _tc_body,
        grid=(nb,),
        in_specs=[
            pl.BlockSpec((MT, g.shape[1]), lambda i: (i, 0)),
            pl.BlockSpec((1, 1, MT), lambda i: (i, 0, 0)),
            pl.BlockSpec((1, 1, MT), lambda i: (i, 0, 0)),
            pl.BlockSpec((1, 1, MT), lambda i: (i, 0, 0)),
            pl.BlockSpec((80, 128), lambda i: (0, 0)),
            pl.BlockSpec((1, 128), lambda i: (0, 0)),
            pl.BlockSpec((128, 16), lambda i: (0, 0)),
        ],
        out_specs=pl.BlockSpec((mb, L, 128), lambda i: (i, 0, 0)),
        out_shape=jax.ShapeDtypeStruct((B, L, 128), jnp.float32),
    )(g, r3, v3, q3, w, b2, small_cat)


def kernel(urls, ranks, vtypes, q_iter, url_table, rank_table, vtype_table,
           qcnt_table, W, b):
    B, L = urls.shape
    n = B * L
    d = url_table.shape[1]
    idx2 = urls.reshape(n // IDX_W, IDX_W).astype(jnp.int32)
    g = _make_sc_gather(n, d)(idx2, url_table)

    small_cat = jnp.zeros((128, 16), jnp.float32)
    small_cat = small_cat.at[0:11, 0:4].set(rank_table)
    small_cat = small_cat.at[11:111, 4:12].set(vtype_table)
    small_cat = small_cat.at[111:122, 12:16].set(qcnt_table)

    nb = n // MT
    r3 = ranks.reshape(nb, 1, MT).astype(jnp.int32)
    v3 = vtypes.reshape(nb, 1, MT).astype(jnp.int32)
    q3 = q_iter.reshape(nb, 1, MT).astype(jnp.int32)
    return _tc_call(g, r3, v3, q3, W, b.reshape(1, 128), small_cat, B, L)


# speedup vs baseline: 11.4501x; 1.1269x over previous
import functools

import jax
import jax.numpy as jnp
from jax import lax
from jax.experimental import pallas as pl
from jax.experimental.pallas import tpu as pltpu
from jax.experimental.pallas import tpu_sc as plsc

NC, NS = 2, 16
NW = NC * NS
IDX_W = 128
T = 1024
HT = 512
MT = 1600
D = 128


def _make_sc_gather(n_tokens):
    tpw = n_tokens // NW
    steps = tpw // T
    mesh = plsc.VectorSubcoreMesh(core_axis_name="c", subcore_axis_name="s")

    @functools.partial(
        pl.kernel,
        out_type=jax.ShapeDtypeStruct((n_tokens, D), jnp.float32),
        mesh=mesh,
        compiler_params=pltpu.CompilerParams(use_tc_tiling_on_sc=True),
        scratch_types=[
            pltpu.VMEM((T // IDX_W, IDX_W), jnp.int32),
            pltpu.VMEM((HT, D), jnp.float32),
            pltpu.SemaphoreType.DMA,
        ],
    )
    def sc_gather(idx_hbm, table_hbm, out_hbm, idx_v, rows_v, sem):
        cid = lax.axis_index("c")
        sid = lax.axis_index("s")
        wid = sid * NC + cid
        row_base = wid * (tpw // IDX_W)

        def body(step, carry):
            row_off = row_base + step * (T // IDX_W)
            pltpu.sync_copy(idx_hbm.at[pl.ds(row_off, T // IDX_W)], idx_v)
            for h in range(T // HT):
                cps = [
                    pltpu.async_copy(
                        table_hbm.at[idx_v.at[h * (HT // IDX_W) + j]],
                        rows_v.at[pl.ds(j * IDX_W, IDX_W)], sem)
                    for j in range(HT // IDX_W)
                ]
                for cp in cps:
                    cp.wait()
                pltpu.sync_copy(
                    rows_v, out_hbm.at[pl.ds(row_off * IDX_W + h * HT, HT)])
            return carry

        lax.fori_loop(0, steps, body, 0)

    return sc_gather


def _proj_body(t_ref, w_ref, o_ref):
    o_ref[...] = jnp.dot(t_ref[...], w_ref[...][0:64, :],
                         preferred_element_type=jnp.float32)


def _project_table(url_table, w):
    v, du = url_table.shape
    rb = 1000
    return pl.pallas_call(
        _proj_body,
        grid=(v // rb,),
        in_specs=[
            pl.BlockSpec((rb, du), lambda i: (i, 0)),
            pl.BlockSpec((80, 128), lambda i: (0, 0)),
        ],
        out_specs=pl.BlockSpec((rb, D), lambda i: (i, 0)),
        out_shape=jax.ShapeDtypeStruct((v, D), jnp.float32),
    )(url_table, w)


def _tc_body(g_ref, r_ref, v_ref, q_ref, w_ref, b_ref, cat_ref, o_ref):
    w_small = w_ref[...][64:80, :]
    small_proj = jnp.dot(cat_ref[...], w_small,
                         preferred_element_type=jnp.float32)
    r = r_ref[0]
    v = v_ref[0]
    q = q_ref[0]
    ji = lax.broadcasted_iota(jnp.int32, (128, MT), 0)
    mh = (ji == r) | (ji == v + 11) | (ji == q + 111)
    mhf = jnp.where(mh, 1.0, 0.0).astype(jnp.float32)
    small = lax.dot_general(mhf, small_proj, (((0,), (0,)), ((), ())),
                            preferred_element_type=jnp.float32)
    res = jnp.tanh(g_ref[...] + small + b_ref[...])
    o_ref[...] = res.reshape(o_ref.shape)


def _tc_call(g, r3, v3, q3, w, b2, small_cat, B, L):
    n = g.shape[0]
    nb = n // MT
    mb = MT // L
    return pl.pallas_call(
        _tc_body,
        grid=(nb,),
        in_specs=[
            pl.BlockSpec((MT, D), lambda i: (i, 0)),
            pl.BlockSpec((1, 1, MT), lambda i: (i, 0, 0)),
            pl.BlockSpec((1, 1, MT), lambda i: (i, 0, 0)),
            pl.BlockSpec((1, 1, MT), lambda i: (i, 0, 0)),
            pl.BlockSpec((80, 128), lambda i: (0, 0)),
            pl.BlockSpec((1, 128), lambda i: (0, 0)),
            pl.BlockSpec((128, 16), lambda i: (0, 0)),
        ],
        out_specs=pl.BlockSpec((mb, L, 128), lambda i: (i, 0, 0)),
        out_shape=jax.ShapeDtypeStruct((B, L, 128), jnp.float32),
    )(g, r3, v3, q3, w, b2, small_cat)


def kernel(urls, ranks, vtypes, q_iter, url_table, rank_table, vtype_table,
           qcnt_table, W, b):
    B, L = urls.shape
    n = B * L
    url_proj = _project_table(url_table, W)
    idx2 = urls.reshape(n // IDX_W, IDX_W).astype(jnp.int32)
    g = _make_sc_gather(n)(idx2, url_proj)

    small_cat = jnp.zeros((128, 16), jnp.float32)
    small_cat = small_cat.at[0:11, 0:4].set(rank_table)
    small_cat = small_cat.at[11:111, 4:12].set(vtype_table)
    small_cat = small_cat.at[111:122, 12:16].set(qcnt_table)

    nb = n // MT
    r3 = ranks.reshape(nb, 1, MT).astype(jnp.int32)
    v3 = vtypes.reshape(nb, 1, MT).astype(jnp.int32)
    q3 = q_iter.reshape(nb, 1, MT).astype(jnp.int32)
    return _tc_call(g, r3, v3, q3, W, b.reshape(1, 128), small_cat, B, L)


# speedup vs baseline: 12.2046x; 1.0659x over previous
import functools

import jax
import jax.numpy as jnp
from jax import lax
from jax.experimental import pallas as pl
from jax.experimental.pallas import tpu as pltpu
from jax.experimental.pallas import tpu_sc as plsc

NC, NS = 2, 16
NW = NC * NS
IDX_W = 128
T = 1024
HT = 512
MT = 2048
D = 128


def _make_sc_gather(n_tokens):
    tpw = n_tokens // NW
    steps = tpw // T
    mesh = plsc.VectorSubcoreMesh(core_axis_name="c", subcore_axis_name="s")

    @functools.partial(
        pl.kernel,
        out_type=jax.ShapeDtypeStruct((n_tokens, D), jnp.float32),
        mesh=mesh,
        compiler_params=pltpu.CompilerParams(use_tc_tiling_on_sc=True),
        scratch_types=[
            pltpu.VMEM((T // IDX_W, IDX_W), jnp.int32),
            pltpu.VMEM((HT, D), jnp.float32),
            pltpu.SemaphoreType.DMA,
        ],
    )
    def sc_gather(idx_hbm, table_hbm, out_hbm, idx_v, rows_v, sem):
        cid = lax.axis_index("c")
        sid = lax.axis_index("s")
        wid = sid * NC + cid
        row_base = wid * (tpw // IDX_W)

        def body(step, carry):
            row_off = row_base + step * (T // IDX_W)
            pltpu.sync_copy(idx_hbm.at[pl.ds(row_off, T // IDX_W)], idx_v)
            for h in range(T // HT):
                cps = [
                    pltpu.async_copy(
                        table_hbm.at[idx_v.at[h * (HT // IDX_W) + j]],
                        rows_v.at[pl.ds(j * IDX_W, IDX_W)], sem)
                    for j in range(HT // IDX_W)
                ]
                for cp in cps:
                    cp.wait()
                pltpu.sync_copy(
                    rows_v, out_hbm.at[pl.ds(row_off * IDX_W + h * HT, HT)])
            return carry

        lax.fori_loop(0, steps, body, 0)

    return sc_gather


def _proj_body(t_ref, w_ref, o_ref):
    o_ref[...] = jnp.dot(t_ref[...], w_ref[...][0:64, :],
                         preferred_element_type=jnp.float32)


def _project_table(url_table, w):
    v, du = url_table.shape
    rb = 1000
    return pl.pallas_call(
        _proj_body,
        grid=(v // rb,),
        in_specs=[
            pl.BlockSpec((rb, du), lambda i: (i, 0)),
            pl.BlockSpec((80, 128), lambda i: (0, 0)),
        ],
        out_specs=pl.BlockSpec((rb, D), lambda i: (i, 0)),
        out_shape=jax.ShapeDtypeStruct((v, D), jnp.float32),
    )(url_table, w)


def _tc_body(g_ref, r_ref, v_ref, q_ref, w_ref, b_ref, cat_ref, o_ref):
    w_small = w_ref[...][64:80, :]
    small_proj = jnp.dot(cat_ref[...], w_small,
                         preferred_element_type=jnp.float32)
    r = r_ref[0]
    v = v_ref[0]
    q = q_ref[0]
    ji = lax.broadcasted_iota(jnp.int32, (128, MT), 0)
    mh = (ji == r) | (ji == v + 11) | (ji == q + 111)
    mhf = jnp.where(mh, 1.0, 0.0).astype(jnp.float32)
    small = lax.dot_general(mhf, small_proj, (((0,), (0,)), ((), ())),
                            preferred_element_type=jnp.float32)
    o_ref[...] = jnp.tanh(g_ref[...] + small + b_ref[...])


def _tc_call(g, r3, v3, q3, w, b2, small_cat):
    n = g.shape[0]
    nb = n // MT
    return pl.pallas_call(
        _tc_body,
        grid=(nb,),
        in_specs=[
            pl.BlockSpec((MT, D), lambda i: (i, 0)),
            pl.BlockSpec((1, 1, MT), lambda i: (i, 0, 0)),
            pl.BlockSpec((1, 1, MT), lambda i: (i, 0, 0)),
            pl.BlockSpec((1, 1, MT), lambda i: (i, 0, 0)),
            pl.BlockSpec((80, 128), lambda i: (0, 0)),
            pl.BlockSpec((1, 128), lambda i: (0, 0)),
            pl.BlockSpec((128, 16), lambda i: (0, 0)),
        ],
        out_specs=pl.BlockSpec((MT, 128), lambda i: (i, 0)),
        out_shape=jax.ShapeDtypeStruct((n, 128), jnp.float32),
    )(g, r3, v3, q3, w, b2, small_cat)


def kernel(urls, ranks, vtypes, q_iter, url_table, rank_table, vtype_table,
           qcnt_table, W, b):
    B, L = urls.shape
    n = B * L
    url_proj = _project_table(url_table, W)
    idx2 = urls.reshape(n // IDX_W, IDX_W).astype(jnp.int32)
    g = _make_sc_gather(n)(idx2, url_proj)

    small_cat = jnp.zeros((128, 16), jnp.float32)
    small_cat = small_cat.at[0:11, 0:4].set(rank_table)
    small_cat = small_cat.at[11:111, 4:12].set(vtype_table)
    small_cat = small_cat.at[111:122, 12:16].set(qcnt_table)

    nb = n // MT
    r3 = ranks.reshape(nb, 1, MT).astype(jnp.int32)
    v3 = vtypes.reshape(nb, 1, MT).astype(jnp.int32)
    q3 = q_iter.reshape(nb, 1, MT).astype(jnp.int32)
    out = _tc_call(g, r3, v3, q3, W, b.reshape(1, 128), small_cat)
    return out.reshape(B, L, 128)


# speedup vs baseline: 13.8102x; 1.1316x over previous
import functools

import jax
import jax.numpy as jnp
from jax import lax
from jax.experimental import pallas as pl
from jax.experimental.pallas import tpu as pltpu
from jax.experimental.pallas import tpu_sc as plsc

NC, NS = 2, 16
NW = NC * NS
IDX_W = 128
T = 1024
HT = 512
MT = 2048
D = 128


def _make_sc_gather(n_tokens):
    tpw = n_tokens // NW
    steps = tpw // T
    mesh = plsc.VectorSubcoreMesh(core_axis_name="c", subcore_axis_name="s")

    @functools.partial(
        pl.kernel,
        out_type=jax.ShapeDtypeStruct((n_tokens, D), jnp.float32),
        mesh=mesh,
        compiler_params=pltpu.CompilerParams(use_tc_tiling_on_sc=True),
        scratch_types=[
            pltpu.VMEM((T // IDX_W, IDX_W), jnp.int32),
            pltpu.VMEM((HT, D), jnp.float32),
            pltpu.SemaphoreType.DMA,
        ],
    )
    def sc_gather(idx_hbm, table_hbm, out_hbm, idx_v, rows_v, sem):
        cid = lax.axis_index("c")
        sid = lax.axis_index("s")
        wid = sid * NC + cid
        row_base = wid * (tpw // IDX_W)

        def body(step, carry):
            row_off = row_base + step * (T // IDX_W)
            pltpu.sync_copy(idx_hbm.at[pl.ds(row_off, T // IDX_W)], idx_v)
            for h in range(T // HT):
                cps = [
                    pltpu.async_copy(
                        table_hbm.at[idx_v.at[h * (HT // IDX_W) + j]],
                        rows_v.at[pl.ds(j * IDX_W, IDX_W)], sem)
                    for j in range(HT // IDX_W)
                ]
                for cp in cps:
                    cp.wait()
                pltpu.sync_copy(
                    rows_v, out_hbm.at[pl.ds(row_off * IDX_W + h * HT, HT)])
            return carry

        lax.fori_loop(0, steps, body, 0)

    return sc_gather


def _proj_body(t_ref, w_ref, o_ref):
    o_ref[...] = jnp.dot(t_ref[...], w_ref[...][0:64, :],
                         preferred_element_type=jnp.float32)


def _project_table(url_table, w):
    v, du = url_table.shape
    rb = 1000
    return pl.pallas_call(
        _proj_body,
        grid=(v // rb,),
        in_specs=[
            pl.BlockSpec((rb, du), lambda i: (i, 0)),
            pl.BlockSpec((80, 128), lambda i: (0, 0)),
        ],
        out_specs=pl.BlockSpec((rb, D), lambda i: (i, 0)),
        out_shape=jax.ShapeDtypeStruct((v, D), jnp.float32),
    )(url_table, w)


def _tc_body(g_ref, r_ref, v_ref, q_ref, w_ref, b_ref, cat_ref, o_ref):
    w_small = w_ref[...][64:80, :]
    small_proj = jnp.dot(cat_ref[...], w_small,
                         preferred_element_type=jnp.float32)
    r = r_ref[0]
    v = v_ref[0]
    q = q_ref[0]
    ji = lax.broadcasted_iota(jnp.int32, (128, MT), 0)
    mh = (ji == r) | (ji == v + 11) | (ji == q + 111)
    mhf = jnp.where(mh, 1.0, 0.0).astype(jnp.float32)
    small = lax.dot_general(mhf, small_proj, (((0,), (0,)), ((), ())),
                            preferred_element_type=jnp.float32)
    o_ref[...] = jnp.tanh(g_ref[...] + small + b_ref[...])


def _tc_body_prev(prev_ref, g_ref, r_ref, v_ref, q_ref, w_ref, b_ref,
                  cat_ref, o_ref):
    del prev_ref
    _tc_body(g_ref, r_ref, v_ref, q_ref, w_ref, b_ref, cat_ref, o_ref)


def _tc_call_slab(prev, g_s, r3, v3, q3, w, b2, small_cat, tile_off, n):
    nbs = g_s.shape[0] // MT
    data_specs = [
        pl.BlockSpec((MT, D), lambda i: (i, 0)),
        pl.BlockSpec((1, 1, MT), lambda i: (i, 0, 0)),
        pl.BlockSpec((1, 1, MT), lambda i: (i, 0, 0)),
        pl.BlockSpec((1, 1, MT), lambda i: (i, 0, 0)),
        pl.BlockSpec((80, 128), lambda i: (0, 0)),
        pl.BlockSpec((1, 128), lambda i: (0, 0)),
        pl.BlockSpec((128, 16), lambda i: (0, 0)),
    ]
    out_spec = pl.BlockSpec((MT, 128), lambda i: (i + tile_off, 0))
    out_shape = jax.ShapeDtypeStruct((n, 128), jnp.float32)
    if prev is None:
        return pl.pallas_call(
            _tc_body, grid=(nbs,), in_specs=data_specs,
            out_specs=out_spec, out_shape=out_shape,
        )(g_s, r3, v3, q3, w, b2, small_cat)
    return pl.pallas_call(
        _tc_body_prev, grid=(nbs,),
        in_specs=[pl.BlockSpec(memory_space=pl.ANY)] + data_specs,
        out_specs=out_spec, out_shape=out_shape,
        input_output_aliases={0: 0},
    )(prev, g_s, r3, v3, q3, w, b2, small_cat)


def kernel(urls, ranks, vtypes, q_iter, url_table, rank_table, vtype_table,
           qcnt_table, W, b):
    B, L = urls.shape
    n = B * L
    n_slab = 5
    sn = n // n_slab
    url_proj = _project_table(url_table, W)
    idx2 = urls.reshape(n // IDX_W, IDX_W).astype(jnp.int32)
    sc_gather = _make_sc_gather(sn)
    g_slabs = [
        sc_gather(idx2[s * (sn // IDX_W):(s + 1) * (sn // IDX_W)], url_proj)
        for s in range(n_slab)
    ]

    small_cat = jnp.zeros((128, 16), jnp.float32)
    small_cat = small_cat.at[0:11, 0:4].set(rank_table)
    small_cat = small_cat.at[11:111, 4:12].set(vtype_table)
    small_cat = small_cat.at[111:122, 12:16].set(qcnt_table)

    nb = n // MT
    nbs = sn // MT
    r3 = ranks.reshape(nb, 1, MT).astype(jnp.int32)
    v3 = vtypes.reshape(nb, 1, MT).astype(jnp.int32)
    q3 = q_iter.reshape(nb, 1, MT).astype(jnp.int32)
    b2 = b.reshape(1, 128)
    out = None
    for s in range(n_slab):
        sl = slice(s * nbs, (s + 1) * nbs)
        out = _tc_call_slab(out, g_slabs[s], r3[sl], v3[sl], q3[sl],
                            W, b2, small_cat, s * nbs, n)
    return out.reshape(B, L, 128)


# speedup vs baseline: 14.1724x; 1.0262x over previous
import functools

import jax
import jax.numpy as jnp
from jax import lax
from jax.experimental import pallas as pl
from jax.experimental.pallas import tpu as pltpu
from jax.experimental.pallas import tpu_sc as plsc

NC, NS = 2, 16
NW = NC * NS
IDX_W = 128
T = 1024
HT = 512
MT = 2048
D = 128


def _make_sc_gather(n_tokens):
    tpw = n_tokens // NW
    steps = tpw // T
    mesh = plsc.VectorSubcoreMesh(core_axis_name="c", subcore_axis_name="s")

    @functools.partial(
        pl.kernel,
        out_type=jax.ShapeDtypeStruct((n_tokens, D), jnp.float32),
        mesh=mesh,
        compiler_params=pltpu.CompilerParams(use_tc_tiling_on_sc=True),
        scratch_types=[
            pltpu.VMEM((T // IDX_W, IDX_W), jnp.int32),
            pltpu.VMEM((HT, D), jnp.float32),
            pltpu.SemaphoreType.DMA,
        ],
    )
    def sc_gather(idx_hbm, table_hbm, out_hbm, idx_v, rows_v, sem):
        cid = lax.axis_index("c")
        sid = lax.axis_index("s")
        wid = sid * NC + cid
        row_base = wid * (tpw // IDX_W)

        def body(step, carry):
            row_off = row_base + step * (T // IDX_W)
            pltpu.sync_copy(idx_hbm.at[pl.ds(row_off, T // IDX_W)], idx_v)
            for h in range(T // HT):
                cps = [
                    pltpu.async_copy(
                        table_hbm.at[idx_v.at[h * (HT // IDX_W) + j]],
                        rows_v.at[pl.ds(j * IDX_W, IDX_W)], sem)
                    for j in range(HT // IDX_W)
                ]
                for cp in cps:
                    cp.wait()
                pltpu.sync_copy(
                    rows_v, out_hbm.at[pl.ds(row_off * IDX_W + h * HT, HT)])
            return carry

        lax.fori_loop(0, steps, body, 0)

    return sc_gather


def _proj_body(t_ref, w_ref, o_ref):
    o_ref[...] = jnp.dot(t_ref[...], w_ref[...][0:64, :],
                         preferred_element_type=jnp.float32)


def _project_table(url_table, w):
    v, du = url_table.shape
    rb = 2000
    return pl.pallas_call(
        _proj_body,
        grid=(v // rb,),
        in_specs=[
            pl.BlockSpec((rb, du), lambda i: (i, 0)),
            pl.BlockSpec((80, 128), lambda i: (0, 0)),
        ],
        out_specs=pl.BlockSpec((rb, D), lambda i: (i, 0)),
        out_shape=jax.ShapeDtypeStruct((v, D), jnp.float32),
    )(url_table, w)


def _tc_body(g_ref, r_ref, v_ref, q_ref, w_ref, b_ref, cat_ref, o_ref):
    w_small = w_ref[...][64:80, :]
    small_proj = jnp.dot(cat_ref[...], w_small,
                         preferred_element_type=jnp.float32)
    r = r_ref[0]
    v = v_ref[0]
    q = q_ref[0]
    ji = lax.broadcasted_iota(jnp.int32, (128, MT), 0)
    mh = (ji == r) | (ji == v + 11) | (ji == q + 111)
    mhf = jnp.where(mh, 1.0, 0.0).astype(jnp.float32)
    small = lax.dot_general(mhf, small_proj, (((0,), (0,)), ((), ())),
                            preferred_element_type=jnp.float32)
    o_ref[...] = jnp.tanh(g_ref[...] + small + b_ref[...])


def _tc_body_prev(prev_ref, g_ref, r_ref, v_ref, q_ref, w_ref, b_ref,
                  cat_ref, o_ref):
    del prev_ref
    _tc_body(g_ref, r_ref, v_ref, q_ref, w_ref, b_ref, cat_ref, o_ref)


def _tc_call_slab(prev, g_s, r3, v3, q3, w, b2, small_cat, tile_off, n):
    nbs = g_s.shape[0] // MT
    data_specs = [
        pl.BlockSpec((MT, D), lambda i: (i, 0)),
        pl.BlockSpec((1, 1, MT), lambda i: (i, 0, 0)),
        pl.BlockSpec((1, 1, MT), lambda i: (i, 0, 0)),
        pl.BlockSpec((1, 1, MT), lambda i: (i, 0, 0)),
        pl.BlockSpec((80, 128), lambda i: (0, 0)),
        pl.BlockSpec((1, 128), lambda i: (0, 0)),
        pl.BlockSpec((128, 16), lambda i: (0, 0)),
    ]
    out_spec = pl.BlockSpec((MT, 128), lambda i: (i + tile_off, 0))
    out_shape = jax.ShapeDtypeStruct((n, 128), jnp.float32)
    if prev is None:
        return pl.pallas_call(
            _tc_body, grid=(nbs,), in_specs=data_specs,
            out_specs=out_spec, out_shape=out_shape,
        )(g_s, r3, v3, q3, w, b2, small_cat)
    return pl.pallas_call(
        _tc_body_prev, grid=(nbs,),
        in_specs=[pl.BlockSpec(memory_space=pl.ANY)] + data_specs,
        out_specs=out_spec, out_shape=out_shape,
        input_output_aliases={0: 0},
    )(prev, g_s, r3, v3, q3, w, b2, small_cat)


def kernel(urls, ranks, vtypes, q_iter, url_table, rank_table, vtype_table,
           qcnt_table, W, b):
    B, L = urls.shape
    n = B * L
    unit = NW * T
    parts = (2, 3, 4, 4, 4, 4, 4)
    url_proj = _project_table(url_table, W)
    idx2 = urls.reshape(n // IDX_W, IDX_W).astype(jnp.int32)
    gather_fns = {u: _make_sc_gather(u * unit) for u in set(parts)}
    g_slabs = []
    row_off = 0
    for u in parts:
        rows = u * unit // IDX_W
        g_slabs.append(
            gather_fns[u](idx2[row_off:row_off + rows], url_proj))
        row_off += rows

    small_cat = jnp.zeros((128, 16), jnp.float32)
    small_cat = small_cat.at[0:11, 0:4].set(rank_table)
    small_cat = small_cat.at[11:111, 4:12].set(vtype_table)
    small_cat = small_cat.at[111:122, 12:16].set(qcnt_table)

    nb = n // MT
    r3 = ranks.reshape(nb, 1, MT).astype(jnp.int32)
    v3 = vtypes.reshape(nb, 1, MT).astype(jnp.int32)
    q3 = q_iter.reshape(nb, 1, MT).astype(jnp.int32)
    b2 = b.reshape(1, 128)
    out = None
    tile_off = 0
    for s, u in enumerate(parts):
        nbs = u * unit // MT
        sl = slice(tile_off, tile_off + nbs)
        out = _tc_call_slab(out, g_slabs[s], r3[sl], v3[sl], q3[sl],
                            W, b2, small_cat, tile_off, n)
        tile_off += nbs
    return out.reshape(B, L, 128)


# speedup vs baseline: 14.6237x; 1.0318x over previous
import functools

import jax
import jax.numpy as jnp
from jax import lax
from jax.experimental import pallas as pl
from jax.experimental.pallas import tpu as pltpu
from jax.experimental.pallas import tpu_sc as plsc

NC, NS = 2, 16
NW = NC * NS
IDX_W = 128
T = 1024
HT = 512
MT = 2048
D = 128


def _make_sc_gather(n_tokens):
    tpw = n_tokens // NW
    steps = tpw // T
    mesh = plsc.VectorSubcoreMesh(core_axis_name="c", subcore_axis_name="s")

    @functools.partial(
        pl.kernel,
        out_type=jax.ShapeDtypeStruct((n_tokens, D), jnp.float32),
        mesh=mesh,
        compiler_params=pltpu.CompilerParams(use_tc_tiling_on_sc=True),
        scratch_types=[
            pltpu.VMEM((T // IDX_W, IDX_W), jnp.int32),
            pltpu.VMEM((HT, D), jnp.float32),
            pltpu.SemaphoreType.DMA,
        ],
    )
    def sc_gather(idx_hbm, table_hbm, out_hbm, idx_v, rows_v, sem):
        cid = lax.axis_index("c")
        sid = lax.axis_index("s")
        wid = sid * NC + cid
        row_base = wid * (tpw // IDX_W)

        def body(step, carry):
            row_off = row_base + step * (T // IDX_W)
            pltpu.sync_copy(idx_hbm.at[pl.ds(row_off, T // IDX_W)], idx_v)
            for h in range(T // HT):
                cps = [
                    pltpu.async_copy(
                        table_hbm.at[idx_v.at[h * (HT // IDX_W) + j]],
                        rows_v.at[pl.ds(j * IDX_W, IDX_W)], sem)
                    for j in range(HT // IDX_W)
                ]
                for cp in cps:
                    cp.wait()
                pltpu.sync_copy(
                    rows_v, out_hbm.at[pl.ds(row_off * IDX_W + h * HT, HT)])
            return carry

        lax.fori_loop(0, steps, body, 0)

    return sc_gather


def _proj_body(t_ref, w_ref, o_ref):
    o_ref[...] = jnp.dot(t_ref[...], w_ref[...][0:64, :],
                         preferred_element_type=jnp.float32)


def _project_table(url_table, w):
    v, du = url_table.shape
    rb = 2000
    return pl.pallas_call(
        _proj_body,
        grid=(v // rb,),
        in_specs=[
            pl.BlockSpec((rb, du), lambda i: (i, 0)),
            pl.BlockSpec((80, 128), lambda i: (0, 0)),
        ],
        out_specs=pl.BlockSpec((rb, D), lambda i: (i, 0)),
        out_shape=jax.ShapeDtypeStruct((v, D), jnp.float32),
    )(url_table, w)


def _tc_body(g_ref, c_ref, w_ref, b_ref, cat_ref, o_ref):
    w_small = w_ref[...][64:80, :]
    small_proj = jnp.dot(cat_ref[...], w_small,
                         preferred_element_type=jnp.float32)
    code = c_ref[0]
    r = code & 15
    v = (code >> 4) & 127
    q = code >> 11
    ji = lax.broadcasted_iota(jnp.int32, (128, MT), 0)
    mh = (ji == r) | (ji == v + 11) | (ji == q + 111)
    mhf = jnp.where(mh, 1.0, 0.0).astype(jnp.float32)
    small = lax.dot_general(mhf, small_proj, (((0,), (0,)), ((), ())),
                            preferred_element_type=jnp.float32)
    o_ref[...] = jnp.tanh(g_ref[...] + small + b_ref[...])


def _tc_body_prev(prev_ref, g_ref, c_ref, w_ref, b_ref, cat_ref, o_ref):
    del prev_ref
    _tc_body(g_ref, c_ref, w_ref, b_ref, cat_ref, o_ref)


def _tc_call_slab(prev, g_s, c3, w, b2, small_cat, tile_off, n):
    nbs = g_s.shape[0] // MT
    data_specs = [
        pl.BlockSpec((MT, D), lambda i: (i, 0)),
        pl.BlockSpec((1, 1, MT), lambda i: (i, 0, 0)),
        pl.BlockSpec((80, 128), lambda i: (0, 0)),
        pl.BlockSpec((1, 128), lambda i: (0, 0)),
        pl.BlockSpec((128, 16), lambda i: (0, 0)),
    ]
    out_spec = pl.BlockSpec((MT, 128), lambda i: (i + tile_off, 0))
    out_shape = jax.ShapeDtypeStruct((n, 128), jnp.float32)
    if prev is None:
        return pl.pallas_call(
            _tc_body, grid=(nbs,), in_specs=data_specs,
            out_specs=out_spec, out_shape=out_shape,
        )(g_s, c3, w, b2, small_cat)
    return pl.pallas_call(
        _tc_body_prev, grid=(nbs,),
        in_specs=[pl.BlockSpec(memory_space=pl.ANY)] + data_specs,
        out_specs=out_spec, out_shape=out_shape,
        input_output_aliases={0: 0},
    )(prev, g_s, c3, w, b2, small_cat)


def kernel(urls, ranks, vtypes, q_iter, url_table, rank_table, vtype_table,
           qcnt_table, W, b):
    B, L = urls.shape
    n = B * L
    unit = NW * T
    parts = (2, 3, 4, 4, 4, 4, 4)
    url_proj = _project_table(url_table, W)
    idx2 = urls.reshape(n // IDX_W, IDX_W).astype(jnp.int32)
    gather_fns = {u: _make_sc_gather(u * unit) for u in set(parts)}
    g_slabs = []
    row_off = 0
    for u in parts:
        rows = u * unit // IDX_W
        g_slabs.append(
            gather_fns[u](idx2[row_off:row_off + rows], url_proj))
        row_off += rows

    small_cat = jnp.zeros((128, 16), jnp.float32)
    small_cat = small_cat.at[0:11, 0:4].set(rank_table)
    small_cat = small_cat.at[11:111, 4:12].set(vtype_table)
    small_cat = small_cat.at[111:122, 12:16].set(qcnt_table)

    nb = n // MT
    code = (ranks | (vtypes << 4) | (q_iter << 11)).astype(jnp.int32)
    c3 = code.reshape(nb, 1, MT)
    b2 = b.reshape(1, 128)
    out = None
    tile_off = 0
    for s, u in enumerate(parts):
        nbs = u * unit // MT
        sl = slice(tile_off, tile_off + nbs)
        out = _tc_call_slab(out, g_slabs[s], c3[sl],
                            W, b2, small_cat, tile_off, n)
        tile_off += nbs
    return out.reshape(B, L, 128)


# speedup vs baseline: 16.4002x; 1.1215x over previous
import functools

import jax
import jax.numpy as jnp
from jax import lax
from jax.experimental import pallas as pl
from jax.experimental.pallas import tpu as pltpu
from jax.experimental.pallas import tpu_sc as plsc

NC, NS = 2, 16
NW = NC * NS
IDX_W = 128
T = 1024
HT = 512
MT = 4096
D = 128


def _make_sc_gather(n_tokens):
    tpw = n_tokens // NW
    steps = tpw // T
    mesh = plsc.VectorSubcoreMesh(core_axis_name="c", subcore_axis_name="s")

    @functools.partial(
        pl.kernel,
        out_type=jax.ShapeDtypeStruct((n_tokens, D), jnp.float32),
        mesh=mesh,
        compiler_params=pltpu.CompilerParams(use_tc_tiling_on_sc=True),
        scratch_types=[
            pltpu.VMEM((T // IDX_W, IDX_W), jnp.int32),
            pltpu.VMEM((HT, D), jnp.float32),
            pltpu.SemaphoreType.DMA,
        ],
    )
    def sc_gather(idx_hbm, table_hbm, out_hbm, idx_v, rows_v, sem):
        cid = lax.axis_index("c")
        sid = lax.axis_index("s")
        wid = sid * NC + cid
        row_base = wid * (tpw // IDX_W)

        def body(step, carry):
            row_off = row_base + step * (T // IDX_W)
            pltpu.sync_copy(idx_hbm.at[pl.ds(row_off, T // IDX_W)], idx_v)
            for h in range(T // HT):
                cps = [
                    pltpu.async_copy(
                        table_hbm.at[idx_v.at[h * (HT // IDX_W) + j]],
                        rows_v.at[pl.ds(j * IDX_W, IDX_W)], sem)
                    for j in range(HT // IDX_W)
                ]
                for cp in cps:
                    cp.wait()
                pltpu.sync_copy(
                    rows_v, out_hbm.at[pl.ds(row_off * IDX_W + h * HT, HT)])
            return carry

        lax.fori_loop(0, steps, body, 0)

    return sc_gather


def _proj_body(t_ref, w_ref, o_ref):
    o_ref[...] = jnp.dot(t_ref[...], w_ref[...][0:64, :],
                         preferred_element_type=jnp.float32)


def _project_table(url_table, w):
    v, du = url_table.shape
    rb = 4000
    return pl.pallas_call(
        _proj_body,
        grid=(v // rb,),
        in_specs=[
            pl.BlockSpec((rb, du), lambda i: (i, 0)),
            pl.BlockSpec((80, 128), lambda i: (0, 0)),
        ],
        out_specs=pl.BlockSpec((rb, D), lambda i: (i, 0)),
        out_shape=jax.ShapeDtypeStruct((v, D), jnp.float32),
    )(url_table, w)


def _tc_body(g_ref, c_ref, w_ref, b_ref, cat_ref, o_ref):
    w_small = w_ref[...][64:80, :]
    small_proj = jnp.dot(cat_ref[...], w_small,
                         preferred_element_type=jnp.float32)
    code = c_ref[0]
    r = code & 15
    v = (code >> 4) & 127
    q = code >> 11
    ji = lax.broadcasted_iota(jnp.int32, (128, MT), 0)
    mh = (ji == r) | (ji == v + 11) | (ji == q + 111)
    mhf = jnp.where(mh, 1.0, 0.0).astype(jnp.float32)
    small = lax.dot_general(mhf, small_proj, (((0,), (0,)), ((), ())),
                            preferred_element_type=jnp.float32)
    o_ref[...] = jnp.tanh(g_ref[...] + small + b_ref[...])


def _tc_body_prev(prev_ref, g_ref, c_ref, w_ref, b_ref, cat_ref, o_ref):
    del prev_ref
    _tc_body(g_ref, c_ref, w_ref, b_ref, cat_ref, o_ref)


def _tc_call_slab(prev, g_s, c3, w, b2, small_cat, tile_off, n):
    nbs = g_s.shape[0] // MT
    data_specs = [
        pl.BlockSpec((MT, D), lambda i: (i, 0)),
        pl.BlockSpec((1, 1, MT), lambda i: (i, 0, 0)),
        pl.BlockSpec((80, 128), lambda i: (0, 0)),
        pl.BlockSpec((1, 128), lambda i: (0, 0)),
        pl.BlockSpec((128, 16), lambda i: (0, 0)),
    ]
    out_spec = pl.BlockSpec((MT, 128), lambda i: (i + tile_off, 0))
    out_shape = jax.ShapeDtypeStruct((n, 128), jnp.float32)
    if prev is None:
        return pl.pallas_call(
            _tc_body, grid=(nbs,), in_specs=data_specs,
            out_specs=out_spec, out_shape=out_shape,
        )(g_s, c3, w, b2, small_cat)
    return pl.pallas_call(
        _tc_body_prev, grid=(nbs,),
        in_specs=[pl.BlockSpec(memory_space=pl.ANY)] + data_specs,
        out_specs=out_spec, out_shape=out_shape,
        input_output_aliases={0: 0},
    )(prev, g_s, c3, w, b2, small_cat)


def kernel(urls, ranks, vtypes, q_iter, url_table, rank_table, vtype_table,
           qcnt_table, W, b):
    B, L = urls.shape
    n = B * L
    unit = NW * T
    parts = (2, 3, 4, 4, 4, 4, 4)
    url_proj = _project_table(url_table, W)
    idx2 = urls.reshape(n // IDX_W, IDX_W).astype(jnp.int32)
    gather_fns = {u: _make_sc_gather(u * unit) for u in set(parts)}
    g_slabs = []
    row_off = 0
    for u in parts:
        rows = u * unit // IDX_W
        g_slabs.append(
            gather_fns[u](idx2[row_off:row_off + rows], url_proj))
        row_off += rows

    small_cat = jnp.zeros((128, 16), jnp.float32)
    small_cat = small_cat.at[0:11, 0:4].set(rank_table)
    small_cat = small_cat.at[11:111, 4:12].set(vtype_table)
    small_cat = small_cat.at[111:122, 12:16].set(qcnt_table)

    nb = n // MT
    code = (ranks | (vtypes << 4) | (q_iter << 11)).astype(jnp.int32)
    c3 = code.reshape(nb, 1, MT)
    b2 = b.reshape(1, 128)
    out = None
    tile_off = 0
    for s, u in enumerate(parts):
        nbs = u * unit // MT
        sl = slice(tile_off, tile_off + nbs)
        out = _tc_call_slab(out, g_slabs[s], c3[sl],
                            W, b2, small_cat, tile_off, n)
        tile_off += nbs
    return out.reshape(B, L, 128)
